# baseline (speedup 1.0000x reference)
import functools

import jax
import jax.numpy as jnp
from jax import lax
from jax.experimental import pallas as pl
from jax.experimental.pallas import tpu as pltpu
from jax.experimental.pallas import tpu_sc as plsc

_NC = 2
_NS = 16
_L = 16
_NW = _NC * _NS


def _pose_tc_body(q_ref, c_ref, k_ref, lm_ref, pose_ref, aux_ref, lmr_ref):
    logq = q_ref[:]
    n = jnp.maximum(jnp.sqrt(jnp.sum(logq * logq, axis=1, keepdims=True)), 1e-8)
    vec = logq * jnp.sin(n) / n
    w = jnp.cos(n)
    qn = jnp.maximum(
        jnp.sqrt(w * w + jnp.sum(vec * vec, axis=1, keepdims=True)), 1e-12)
    w = w / qn
    vec = vec / qn
    x = vec[:, 0:1]
    y = vec[:, 1:2]
    z = vec[:, 2:3]
    tx, ty, tz = 2.0 * x, 2.0 * y, 2.0 * z
    twx, twy, twz = tx * w, ty * w, tz * w
    txx, txy, txz = tx * x, ty * x, tz * x
    tyy, tyz = ty * y, tz * y
    tzz = tz * z
    one = jnp.ones_like(x)
    cp = c_ref[:]
    cols = [
        one - (tyy + tzz), txy - twz, txz + twy,
        txy + twz, one - (txx + tzz), tyz - twx,
        txz - twy, tyz + twx, one - (txx + tyy),
        cp[:, 0:1], cp[:, 1:2], cp[:, 2:3],
        jnp.zeros((q_ref.shape[0], 4), jnp.float32),
    ]
    pose = jnp.concatenate(cols, axis=1)
    pose_ref[:] = pose.astype(jnp.bfloat16).astype(jnp.float32)
    lmr_ref[:] = lm_ref[:].astype(jnp.bfloat16).astype(jnp.float32)
    km = k_ref[:]
    aux_ref[:] = jnp.concatenate(
        [
            jnp.broadcast_to(km[0:1, 0:1], (1, 128)),
            jnp.broadcast_to(km[1:2, 1:2], (1, 128)),
            jnp.broadcast_to(km[0:1, 2:3], (1, 128)),
            jnp.broadcast_to(km[1:2, 2:3], (1, 128)),
            jnp.zeros((4, 128), jnp.float32),
        ],
        axis=0,
    )


def _make_pose_tables(quats_log, camera_position, k_mat, landmarks):
    n_kf = quats_log.shape[0]
    return pl.pallas_call(
        _pose_tc_body,
        out_shape=(
            jax.ShapeDtypeStruct((n_kf, 16), jnp.float32),
            jax.ShapeDtypeStruct((8, 128), jnp.float32),
            jax.ShapeDtypeStruct(landmarks.shape, jnp.float32),
        ),
    )(quats_log, camera_position, k_mat, landmarks)


def _make_sc_project(m_obs, n_kf, n_mp):
    chunk = m_obs // _NW
    steps = chunk // _L
    mesh = plsc.VectorSubcoreMesh(core_axis_name="c", subcore_axis_name="s")

    @functools.partial(
        pl.kernel,
        out_type=jax.ShapeDtypeStruct((m_obs, 2), jnp.float32),
        mesh=mesh,
        compiler_params=pltpu.CompilerParams(
            needs_layout_passes=False, use_tc_tiling_on_sc=False),
        scratch_types=[
            pltpu.VMEM((n_kf, 16), jnp.float32),
            pltpu.VMEM((8, 128), jnp.float32),
            pltpu.VMEM((n_mp, 3), jnp.float32),
            pltpu.VMEM((chunk, 1), jnp.int32),
            pltpu.VMEM((chunk, 1), jnp.int32),
            pltpu.VMEM((chunk, 2), jnp.float32),
        ],
    )
    def sc_project(pose_hbm, aux_hbm, lm_hbm, fid_hbm, pid_hbm, out_hbm,
                   pose_v, aux_v, lm_v, fid_v, pid_v, out_v):
        wid = lax.axis_index("s") * _NC + lax.axis_index("c")
        base = wid * chunk
        pltpu.sync_copy(pose_hbm, pose_v)
        pltpu.sync_copy(aux_hbm, aux_v)
        pltpu.sync_copy(lm_hbm, lm_v)
        pltpu.sync_copy(fid_hbm.at[pl.ds(base, chunk)], fid_v)
        pltpu.sync_copy(pid_hbm.at[pl.ds(base, chunk)], pid_v)

        fxv = aux_v[0, pl.ds(0, _L)]
        fyv = aux_v[1, pl.ds(0, _L)]
        cxv = aux_v[2, pl.ds(0, _L)]
        cyv = aux_v[3, pl.ds(0, _L)]
        lane = lax.iota(jnp.int32, _L)
        zero16 = jnp.zeros((_L,), jnp.int32)
        one16 = jnp.ones((_L,), jnp.int32)

        def cvec(k):
            return jnp.full((_L,), k, jnp.int32)

        def step(i, carry):
            rows = i * _L + lane
            fid = plsc.load_gather(fid_v, [rows, zero16])
            pid = plsc.load_gather(pid_v, [rows, zero16])
            r00 = plsc.load_gather(pose_v, [fid, cvec(0)])
            r01 = plsc.load_gather(pose_v, [fid, cvec(1)])
            r02 = plsc.load_gather(pose_v, [fid, cvec(2)])
            r10 = plsc.load_gather(pose_v, [fid, cvec(3)])
            r11 = plsc.load_gather(pose_v, [fid, cvec(4)])
            r12 = plsc.load_gather(pose_v, [fid, cvec(5)])
            r20 = plsc.load_gather(pose_v, [fid, cvec(6)])
            r21 = plsc.load_gather(pose_v, [fid, cvec(7)])
            r22 = plsc.load_gather(pose_v, [fid, cvec(8)])
            tx = plsc.load_gather(pose_v, [fid, cvec(9)])
            ty = plsc.load_gather(pose_v, [fid, cvec(10)])
            tz = plsc.load_gather(pose_v, [fid, cvec(11)])
            px = plsc.load_gather(lm_v, [pid, cvec(0)])
            py = plsc.load_gather(lm_v, [pid, cvec(1)])
            pz = plsc.load_gather(lm_v, [pid, cvec(2)])
            xc = r00 * px + r01 * py + r02 * pz + tx
            yc = r10 * px + r11 * py + r12 * pz + ty
            zc = r20 * px + r21 * py + r22 * pz + tz
            s = jnp.where(jnp.abs(zc) > 1e-8, 1.0 / zc, jnp.ones_like(zc))
            u = (xc * s) * fxv + cxv
            v = (yc * s) * fyv + cyv
            plsc.store_scatter(out_v, [rows, zero16], u)
            plsc.store_scatter(out_v, [rows, one16], v)
            return carry

        lax.fori_loop(0, steps, step, 0)
        pltpu.sync_copy(out_v, out_hbm.at[pl.ds(base, chunk)])

    return sc_project


def kernel(QuatsLog, CameraPosition, Landmarks, K, frame_id, point_id):
    pose, aux, lmr = _make_pose_tables(QuatsLog, CameraPosition, K, Landmarks)
    m_obs = frame_id.shape[0]
    sc_project = _make_sc_project(m_obs, QuatsLog.shape[0], Landmarks.shape[0])
    return sc_project(pose, aux, lmr, frame_id, point_id)


# speedup vs baseline: 1.8885x; 1.8885x over previous
import functools

import jax
import jax.numpy as jnp
from jax import lax
from jax.experimental import pallas as pl
from jax.experimental.pallas import tpu as pltpu
from jax.experimental.pallas import tpu_sc as plsc

_NC = 2
_NS = 16
_L = 16
_NW = _NC * _NS


def _pose_tc_body(q_ref, c_ref, k_ref, lm_ref, pose_ref, aux_ref, lmr_ref):
    logq = q_ref[:]
    n = jnp.maximum(jnp.sqrt(jnp.sum(logq * logq, axis=1, keepdims=True)), 1e-8)
    vec = logq * jnp.sin(n) / n
    w = jnp.cos(n)
    qn = jnp.maximum(
        jnp.sqrt(w * w + jnp.sum(vec * vec, axis=1, keepdims=True)), 1e-12)
    w = w / qn
    vec = vec / qn
    x = vec[:, 0:1]
    y = vec[:, 1:2]
    z = vec[:, 2:3]
    tx, ty, tz = 2.0 * x, 2.0 * y, 2.0 * z
    twx, twy, twz = tx * w, ty * w, tz * w
    txx, txy, txz = tx * x, ty * x, tz * x
    tyy, tyz = ty * y, tz * y
    tzz = tz * z
    one = jnp.ones_like(x)
    cp = c_ref[:]
    cols = [
        one - (tyy + tzz), txy - twz, txz + twy,
        txy + twz, one - (txx + tzz), tyz - twx,
        txz - twy, tyz + twx, one - (txx + tyy),
        cp[:, 0:1], cp[:, 1:2], cp[:, 2:3],
        jnp.zeros((q_ref.shape[0], 4), jnp.float32),
    ]
    pose = jnp.concatenate(cols, axis=1)
    pose_ref[:] = pose.astype(jnp.bfloat16).astype(jnp.float32)
    lmr_ref[:] = lm_ref[:].astype(jnp.bfloat16).astype(jnp.float32)
    km = k_ref[:]
    aux_ref[:] = jnp.concatenate(
        [
            jnp.broadcast_to(km[0:1, 0:1], (1, 128)),
            jnp.broadcast_to(km[1:2, 1:2], (1, 128)),
            jnp.broadcast_to(km[0:1, 2:3], (1, 128)),
            jnp.broadcast_to(km[1:2, 2:3], (1, 128)),
            jnp.zeros((4, 128), jnp.float32),
        ],
        axis=0,
    )


def _make_pose_tables(quats_log, camera_position, k_mat, landmarks):
    n_kf = quats_log.shape[0]
    return pl.pallas_call(
        _pose_tc_body,
        out_shape=(
            jax.ShapeDtypeStruct((n_kf, 16), jnp.float32),
            jax.ShapeDtypeStruct((8, 128), jnp.float32),
            jax.ShapeDtypeStruct(landmarks.shape, jnp.float32),
        ),
    )(quats_log, camera_position, k_mat, landmarks)


def _make_sc_project(m_obs, n_kf, n_mp):
    chunk = m_obs // _NW
    steps = chunk // _L
    mesh = plsc.VectorSubcoreMesh(core_axis_name="c", subcore_axis_name="s")

    @functools.partial(
        pl.kernel,
        out_type=jax.ShapeDtypeStruct((m_obs * 2,), jnp.float32),
        mesh=mesh,
        compiler_params=pltpu.CompilerParams(needs_layout_passes=False),
        scratch_types=[
            pltpu.VMEM((n_kf * 16,), jnp.float32),
            pltpu.VMEM((1024,), jnp.float32),
            pltpu.VMEM((n_mp * 3,), jnp.float32),
            pltpu.VMEM((chunk,), jnp.int32),
            pltpu.VMEM((chunk,), jnp.int32),
            pltpu.VMEM((chunk * 2,), jnp.float32),
            pltpu.SemaphoreType.DMA,
        ],
    )
    def sc_project(pose_hbm, aux_hbm, lm_hbm, fid_hbm, pid_hbm, out_hbm,
                   pose_v, aux_v, lm_v, fid_v, pid_v, out_v, sem):
        wid = lax.axis_index("s") * _NC + lax.axis_index("c")
        base = wid * chunk
        cp1 = pltpu.async_copy(pose_hbm, pose_v, sem)
        cp2 = pltpu.async_copy(aux_hbm, aux_v, sem)
        cp3 = pltpu.async_copy(lm_hbm, lm_v, sem)
        cp4 = pltpu.async_copy(fid_hbm.at[pl.ds(base, chunk)], fid_v, sem)
        cp5 = pltpu.async_copy(pid_hbm.at[pl.ds(base, chunk)], pid_v, sem)
        cp1.wait()
        cp2.wait()
        cp3.wait()
        cp4.wait()
        cp5.wait()

        fxv = aux_v[pl.ds(0 * 128, _L)]
        fyv = aux_v[pl.ds(1 * 128, _L)]
        cxv = aux_v[pl.ds(2 * 128, _L)]
        cyv = aux_v[pl.ds(3 * 128, _L)]
        lane = lax.iota(jnp.int32, _L)

        @plsc.parallel_loop(0, steps, unroll=4)
        def step(i):
            off = i * _L
            fid16 = fid_v[pl.ds(off, _L)] * 16
            pid3 = pid_v[pl.ds(off, _L)] * 3
            r00 = plsc.load_gather(pose_v, [fid16])
            r01 = plsc.load_gather(pose_v, [fid16 + 1])
            r02 = plsc.load_gather(pose_v, [fid16 + 2])
            r10 = plsc.load_gather(pose_v, [fid16 + 3])
            r11 = plsc.load_gather(pose_v, [fid16 + 4])
            r12 = plsc.load_gather(pose_v, [fid16 + 5])
            r20 = plsc.load_gather(pose_v, [fid16 + 6])
            r21 = plsc.load_gather(pose_v, [fid16 + 7])
            r22 = plsc.load_gather(pose_v, [fid16 + 8])
            tx = plsc.load_gather(pose_v, [fid16 + 9])
            ty = plsc.load_gather(pose_v, [fid16 + 10])
            tz = plsc.load_gather(pose_v, [fid16 + 11])
            px = plsc.load_gather(lm_v, [pid3])
            py = plsc.load_gather(lm_v, [pid3 + 1])
            pz = plsc.load_gather(lm_v, [pid3 + 2])
            xc = r00 * px + r01 * py + r02 * pz + tx
            yc = r10 * px + r11 * py + r12 * pz + ty
            zc = r20 * px + r21 * py + r22 * pz + tz
            s = jnp.where(jnp.abs(zc) > 1e-8, 1.0 / zc, jnp.ones_like(zc))
            u = (xc * s) * fxv + cxv
            v = (yc * s) * fyv + cyv
            rows2 = (off + lane) * 2
            plsc.store_scatter(out_v, [rows2], u)
            plsc.store_scatter(out_v, [rows2 + 1], v)

        pltpu.sync_copy(out_v, out_hbm.at[pl.ds(base * 2, chunk * 2)])

    return sc_project


def kernel(QuatsLog, CameraPosition, Landmarks, K, frame_id, point_id):
    pose, aux, lmr = _make_pose_tables(QuatsLog, CameraPosition, K, Landmarks)
    m_obs = frame_id.shape[0]
    fid = frame_id.reshape(m_obs)
    pid = point_id.reshape(m_obs)
    sc_project = _make_sc_project(m_obs, QuatsLog.shape[0], Landmarks.shape[0])
    out_flat = sc_project(pose.reshape(-1), aux.reshape(-1),
                          lmr.reshape(-1), fid, pid)
    return out_flat.reshape(m_obs, 2)


# speedup vs baseline: 2.9313x; 1.5522x over previous
import functools

import jax
import jax.numpy as jnp
from jax import lax
from jax.experimental import pallas as pl
from jax.experimental.pallas import tpu as pltpu
from jax.experimental.pallas import tpu_sc as plsc

_NC = 2
_NS = 16
_L = 16
_NW = _NC * _NS


def _pose_tc_body(q_ref, c_ref, k_ref, lm_ref, pose_ref, aux_ref, lmr_ref):
    logq = q_ref[:]
    n = jnp.maximum(jnp.sqrt(jnp.sum(logq * logq, axis=1, keepdims=True)), 1e-8)
    vec = logq * jnp.sin(n) / n
    w = jnp.cos(n)
    qn = jnp.maximum(
        jnp.sqrt(w * w + jnp.sum(vec * vec, axis=1, keepdims=True)), 1e-12)
    w = w / qn
    vec = vec / qn
    x = vec[:, 0:1]
    y = vec[:, 1:2]
    z = vec[:, 2:3]
    tx, ty, tz = 2.0 * x, 2.0 * y, 2.0 * z
    twx, twy, twz = tx * w, ty * w, tz * w
    txx, txy, txz = tx * x, ty * x, tz * x
    tyy, tyz = ty * y, tz * y
    tzz = tz * z
    one = jnp.ones_like(x)
    cp = c_ref[:]
    cols = [
        one - (tyy + tzz), txy - twz, txz + twy,
        txy + twz, one - (txx + tzz), tyz - twx,
        txz - twy, tyz + twx, one - (txx + tyy),
        cp[:, 0:1], cp[:, 1:2], cp[:, 2:3],
        jnp.zeros((q_ref.shape[0], 4), jnp.float32),
    ]
    pose = jnp.concatenate(cols, axis=1)
    pose_ref[:] = pose.astype(jnp.bfloat16).astype(jnp.float32)
    lmr_ref[:] = lm_ref[:].astype(jnp.bfloat16).astype(jnp.float32)
    km = k_ref[:]
    aux_ref[:] = jnp.concatenate(
        [
            jnp.broadcast_to(km[0:1, 0:1], (1, 128)),
            jnp.broadcast_to(km[1:2, 1:2], (1, 128)),
            jnp.broadcast_to(km[0:1, 2:3], (1, 128)),
            jnp.broadcast_to(km[1:2, 2:3], (1, 128)),
            jnp.zeros((4, 128), jnp.float32),
        ],
        axis=0,
    )


def _make_pose_tables(quats_log, camera_position, k_mat, landmarks):
    n_kf = quats_log.shape[0]
    return pl.pallas_call(
        _pose_tc_body,
        out_shape=(
            jax.ShapeDtypeStruct((n_kf, 16), jnp.float32),
            jax.ShapeDtypeStruct((8, 128), jnp.float32),
            jax.ShapeDtypeStruct(landmarks.shape, jnp.float32),
        ),
    )(quats_log, camera_position, k_mat, landmarks)


def _make_sc_project(m_obs, n_kf, n_mp):
    chunk = m_obs // _NW
    steps = chunk // _L
    mesh = plsc.VectorSubcoreMesh(core_axis_name="c", subcore_axis_name="s")

    @functools.partial(
        pl.kernel,
        out_type=(jax.ShapeDtypeStruct((m_obs,), jnp.float32),
                  jax.ShapeDtypeStruct((m_obs,), jnp.float32)),
        mesh=mesh,
        compiler_params=pltpu.CompilerParams(needs_layout_passes=False),
        scratch_types=[
            pltpu.VMEM((n_kf * 16,), jnp.float32),
            pltpu.VMEM((1024,), jnp.float32),
            pltpu.VMEM((n_mp * 3,), jnp.float32),
            pltpu.VMEM((chunk,), jnp.int32),
            pltpu.VMEM((chunk,), jnp.int32),
            pltpu.VMEM((chunk,), jnp.float32),
            pltpu.VMEM((chunk,), jnp.float32),
            pltpu.SemaphoreType.DMA,
        ],
    )
    def sc_project(pose_hbm, aux_hbm, lm_hbm, fid_hbm, pid_hbm, u_hbm, v_hbm,
                   pose_v, aux_v, lm_v, fid_v, pid_v, u_v, v_v, sem):
        wid = lax.axis_index("s") * _NC + lax.axis_index("c")
        base = wid * chunk
        cp1 = pltpu.async_copy(pose_hbm, pose_v, sem)
        cp2 = pltpu.async_copy(aux_hbm, aux_v, sem)
        cp3 = pltpu.async_copy(lm_hbm, lm_v, sem)
        cp4 = pltpu.async_copy(fid_hbm.at[pl.ds(base, chunk)], fid_v, sem)
        cp5 = pltpu.async_copy(pid_hbm.at[pl.ds(base, chunk)], pid_v, sem)
        cp1.wait()
        cp2.wait()
        cp3.wait()
        cp4.wait()
        cp5.wait()

        fxv = aux_v[pl.ds(0 * 128, _L)]
        fyv = aux_v[pl.ds(1 * 128, _L)]
        cxv = aux_v[pl.ds(2 * 128, _L)]
        cyv = aux_v[pl.ds(3 * 128, _L)]

        @plsc.parallel_loop(0, steps, unroll=4)
        def step(i):
            off = i * _L
            fid16 = fid_v[pl.ds(off, _L)] * 16
            pid3 = pid_v[pl.ds(off, _L)] * 3
            r00 = plsc.load_gather(pose_v, [fid16])
            r01 = plsc.load_gather(pose_v, [fid16 + 1])
            r02 = plsc.load_gather(pose_v, [fid16 + 2])
            r10 = plsc.load_gather(pose_v, [fid16 + 3])
            r11 = plsc.load_gather(pose_v, [fid16 + 4])
            r12 = plsc.load_gather(pose_v, [fid16 + 5])
            r20 = plsc.load_gather(pose_v, [fid16 + 6])
            r21 = plsc.load_gather(pose_v, [fid16 + 7])
            r22 = plsc.load_gather(pose_v, [fid16 + 8])
            tx = plsc.load_gather(pose_v, [fid16 + 9])
            ty = plsc.load_gather(pose_v, [fid16 + 10])
            tz = plsc.load_gather(pose_v, [fid16 + 11])
            px = plsc.load_gather(lm_v, [pid3])
            py = plsc.load_gather(lm_v, [pid3 + 1])
            pz = plsc.load_gather(lm_v, [pid3 + 2])
            xc = r00 * px + r01 * py + r02 * pz + tx
            yc = r10 * px + r11 * py + r12 * pz + ty
            zc = r20 * px + r21 * py + r22 * pz + tz
            s = jnp.where(jnp.abs(zc) > 1e-8, 1.0 / zc, jnp.ones_like(zc))
            u_v[pl.ds(off, _L)] = (xc * s) * fxv + cxv
            v_v[pl.ds(off, _L)] = (yc * s) * fyv + cyv

        pltpu.sync_copy(u_v, u_hbm.at[pl.ds(base, chunk)])
        pltpu.sync_copy(v_v, v_hbm.at[pl.ds(base, chunk)])

    return sc_project


def kernel(QuatsLog, CameraPosition, Landmarks, K, frame_id, point_id):
    pose, aux, lmr = _make_pose_tables(QuatsLog, CameraPosition, K, Landmarks)
    m_obs = frame_id.shape[0]
    fid = frame_id.reshape(m_obs)
    pid = point_id.reshape(m_obs)
    sc_project = _make_sc_project(m_obs, QuatsLog.shape[0], Landmarks.shape[0])
    u, v = sc_project(pose.reshape(-1), aux.reshape(-1),
                      lmr.reshape(-1), fid, pid)
    return jnp.concatenate([u.reshape(m_obs, 1), v.reshape(m_obs, 1)], axis=1)


# speedup vs baseline: 3.9175x; 1.3364x over previous
import functools

import jax
import jax.numpy as jnp
from jax import lax
from jax.experimental import pallas as pl
from jax.experimental.pallas import tpu as pltpu
from jax.experimental.pallas import tpu_sc as plsc

_NC = 2
_NS = 16
_L = 16
_NW = _NC * _NS


def _transpose(x):
    return jax.lax.transpose(x, (1, 0))


def _tables_tc_body(q_ref, c_ref, k_ref, lm_ref, ptab_ref, aux_ref, ltab_ref):
    n_kf = q_ref.shape[0]
    n_blk = n_kf // 128
    for h in range(n_blk):
        qt = _transpose(q_ref[pl.ds(128 * h, 128), :])
        ct = _transpose(c_ref[pl.ds(128 * h, 128), :])
        x = qt[0:1, :]
        y = qt[1:2, :]
        z = qt[2:3, :]
        n = jnp.maximum(jnp.sqrt(x * x + y * y + z * z), 1e-8)
        sn = jnp.sin(n) / n
        qx = x * sn
        qy = y * sn
        qz = z * sn
        qw = jnp.cos(n)
        qn = jnp.maximum(
            jnp.sqrt(qw * qw + qx * qx + qy * qy + qz * qz), 1e-12)
        qw = qw / qn
        qx = qx / qn
        qy = qy / qn
        qz = qz / qn
        tx, ty, tz = 2.0 * qx, 2.0 * qy, 2.0 * qz
        twx, twy, twz = tx * qw, ty * qw, tz * qw
        txx, txy, txz = tx * qx, ty * qx, tz * qx
        tyy, tyz = ty * qy, tz * qy
        tzz = tz * qz
        one = jnp.ones_like(qw)
        rows = jnp.concatenate(
            [
                one - (tyy + tzz), txy - twz, txz + twy,
                txy + twz, one - (txx + tzz), tyz - twx,
                txz - twy, tyz + twx, one - (txx + tyy),
                ct[0:1, :], ct[1:2, :], ct[2:3, :],
                jnp.zeros((4, 128), jnp.float32),
            ],
            axis=0,
        )
        ptab_ref[pl.ds(16 * h, 16), :] = (
            rows.astype(jnp.bfloat16).astype(jnp.float32))

    n_mp = lm_ref.shape[0]
    for h in range(n_mp // 128):
        lt = _transpose(lm_ref[pl.ds(128 * h, 128), :])
        ltab_ref[pl.ds(3 * h, 3), :] = (
            lt.astype(jnp.bfloat16).astype(jnp.float32))

    km = k_ref[:]
    aux_ref[:] = jnp.concatenate(
        [
            jnp.broadcast_to(km[0:1, 0:1], (1, 128)),
            jnp.broadcast_to(km[1:2, 1:2], (1, 128)),
            jnp.broadcast_to(km[0:1, 2:3], (1, 128)),
            jnp.broadcast_to(km[1:2, 2:3], (1, 128)),
            jnp.zeros((4, 128), jnp.float32),
        ],
        axis=0,
    )


def _make_tables(quats_log, camera_position, k_mat, landmarks):
    n_kf = quats_log.shape[0]
    n_mp = landmarks.shape[0]
    return pl.pallas_call(
        _tables_tc_body,
        out_shape=(
            jax.ShapeDtypeStruct((n_kf // 8, 128), jnp.float32),
            jax.ShapeDtypeStruct((8, 128), jnp.float32),
            jax.ShapeDtypeStruct((3 * n_mp // 128, 128), jnp.float32),
        ),
    )(quats_log, camera_position, k_mat, landmarks)


def _make_sc_project(m_obs, n_kf, n_mp):
    chunk = m_obs // _NW
    steps = chunk // _L
    mesh = plsc.VectorSubcoreMesh(core_axis_name="c", subcore_axis_name="s")

    @functools.partial(
        pl.kernel,
        out_type=(jax.ShapeDtypeStruct((m_obs,), jnp.float32),
                  jax.ShapeDtypeStruct((m_obs,), jnp.float32)),
        mesh=mesh,
        compiler_params=pltpu.CompilerParams(needs_layout_passes=False),
        scratch_types=[
            pltpu.VMEM((n_kf // 8, 128), jnp.float32),
            pltpu.VMEM((8, 128), jnp.float32),
            pltpu.VMEM((3 * n_mp // 128, 128), jnp.float32),
            pltpu.VMEM((chunk,), jnp.int32),
            pltpu.VMEM((chunk,), jnp.int32),
            pltpu.VMEM((chunk,), jnp.float32),
            pltpu.VMEM((chunk,), jnp.float32),
            pltpu.SemaphoreType.DMA,
        ],
    )
    def sc_project(ptab_hbm, aux_hbm, ltab_hbm, fid_hbm, pid_hbm, u_hbm, v_hbm,
                   ptab_v, aux_v, ltab_v, fid_v, pid_v, u_v, v_v, sem):
        wid = lax.axis_index("s") * _NC + lax.axis_index("c")
        base = wid * chunk
        cp1 = pltpu.async_copy(ptab_hbm, ptab_v, sem)
        cp2 = pltpu.async_copy(aux_hbm, aux_v, sem)
        cp3 = pltpu.async_copy(ltab_hbm, ltab_v, sem)
        cp4 = pltpu.async_copy(fid_hbm.at[pl.ds(base, chunk)], fid_v, sem)
        cp5 = pltpu.async_copy(pid_hbm.at[pl.ds(base, chunk)], pid_v, sem)
        cp1.wait()
        cp2.wait()
        cp3.wait()
        cp4.wait()
        cp5.wait()

        fxv = aux_v[0, pl.ds(0, _L)]
        fyv = aux_v[1, pl.ds(0, _L)]
        cxv = aux_v[2, pl.ds(0, _L)]
        cyv = aux_v[3, pl.ds(0, _L)]

        def cvec(k):
            return jnp.full((_L,), k, jnp.int32)

        @plsc.parallel_loop(0, steps, unroll=4)
        def step(i):
            off = i * _L
            fid = fid_v[pl.ds(off, _L)]
            pid = pid_v[pl.ds(off, _L)]
            fhi = (fid >> 7) << 4
            flo = fid & 127
            phi = (pid >> 7) * 3
            plo = pid & 127
            r00 = plsc.load_gather(ptab_v, [fhi + cvec(0), flo])
            r01 = plsc.load_gather(ptab_v, [fhi + cvec(1), flo])
            r02 = plsc.load_gather(ptab_v, [fhi + cvec(2), flo])
            r10 = plsc.load_gather(ptab_v, [fhi + cvec(3), flo])
            r11 = plsc.load_gather(ptab_v, [fhi + cvec(4), flo])
            r12 = plsc.load_gather(ptab_v, [fhi + cvec(5), flo])
            r20 = plsc.load_gather(ptab_v, [fhi + cvec(6), flo])
            r21 = plsc.load_gather(ptab_v, [fhi + cvec(7), flo])
            r22 = plsc.load_gather(ptab_v, [fhi + cvec(8), flo])
            tx = plsc.load_gather(ptab_v, [fhi + cvec(9), flo])
            ty = plsc.load_gather(ptab_v, [fhi + cvec(10), flo])
            tz = plsc.load_gather(ptab_v, [fhi + cvec(11), flo])
            px = plsc.load_gather(ltab_v, [phi + cvec(0), plo])
            py = plsc.load_gather(ltab_v, [phi + cvec(1), plo])
            pz = plsc.load_gather(ltab_v, [phi + cvec(2), plo])
            xc = r00 * px + r01 * py + r02 * pz + tx
            yc = r10 * px + r11 * py + r12 * pz + ty
            zc = r20 * px + r21 * py + r22 * pz + tz
            s = jnp.where(jnp.abs(zc) > 1e-8, 1.0 / zc, jnp.ones_like(zc))
            u_v[pl.ds(off, _L)] = (xc * s) * fxv + cxv
            v_v[pl.ds(off, _L)] = (yc * s) * fyv + cyv

        pltpu.sync_copy(u_v, u_hbm.at[pl.ds(base, chunk)])
        pltpu.sync_copy(v_v, v_hbm.at[pl.ds(base, chunk)])

    return sc_project


def kernel(QuatsLog, CameraPosition, Landmarks, K, frame_id, point_id):
    ptab, aux, ltab = _make_tables(QuatsLog, CameraPosition, K, Landmarks)
    m_obs = frame_id.shape[0]
    fid = frame_id.reshape(m_obs)
    pid = point_id.reshape(m_obs)
    sc_project = _make_sc_project(m_obs, QuatsLog.shape[0], Landmarks.shape[0])
    u, v = sc_project(ptab, aux, ltab, fid, pid)
    return jnp.concatenate([u.reshape(m_obs, 1), v.reshape(m_obs, 1)], axis=1)


# speedup vs baseline: 4.0819x; 1.0420x over previous
import functools

import jax
import jax.numpy as jnp
from jax import lax
from jax.experimental import pallas as pl
from jax.experimental.pallas import tpu as pltpu
from jax.experimental.pallas import tpu_sc as plsc

_NC = 2
_NS = 16
_L = 16
_NW = _NC * _NS


def _transpose(x):
    return jax.lax.transpose(x, (1, 0))


def _pack_pairs(lo, hi):
    lo16 = lax.bitcast_convert_type(lo.astype(jnp.bfloat16), jnp.uint16)
    hi16 = lax.bitcast_convert_type(hi.astype(jnp.bfloat16), jnp.uint16)
    w = (hi16.astype(jnp.uint32) << 16) | lo16.astype(jnp.uint32)
    return lax.bitcast_convert_type(w, jnp.int32)


def _tables_tc_body(q_ref, c_ref, k_ref, lm_ref, ptab_ref, aux_ref, ltab_ref):
    n_kf = q_ref.shape[0]
    for h in range(n_kf // 128):
        qt = _transpose(q_ref[pl.ds(128 * h, 128), :])
        ct = _transpose(c_ref[pl.ds(128 * h, 128), :])
        x = qt[0:1, :]
        y = qt[1:2, :]
        z = qt[2:3, :]
        n = jnp.maximum(jnp.sqrt(x * x + y * y + z * z), 1e-8)
        sn = jnp.sin(n) / n
        qx = x * sn
        qy = y * sn
        qz = z * sn
        qw = jnp.cos(n)
        qn = jnp.maximum(
            jnp.sqrt(qw * qw + qx * qx + qy * qy + qz * qz), 1e-12)
        qw = qw / qn
        qx = qx / qn
        qy = qy / qn
        qz = qz / qn
        tx, ty, tz = 2.0 * qx, 2.0 * qy, 2.0 * qz
        twx, twy, twz = tx * qw, ty * qw, tz * qw
        txx, txy, txz = tx * qx, ty * qx, tz * qx
        tyy, tyz = ty * qy, tz * qy
        tzz = tz * qz
        one = jnp.ones_like(qw)
        m00 = one - (tyy + tzz)
        m01 = txy - twz
        m02 = txz + twy
        m10 = txy + twz
        m11 = one - (txx + tzz)
        m12 = tyz - twx
        m20 = txz - twy
        m21 = tyz + twx
        m22 = one - (txx + tyy)
        los = jnp.concatenate(
            [m00, m02, m11, m20, m22, ct[1:2, :]], axis=0)
        his = jnp.concatenate(
            [m01, m10, m12, m21, ct[0:1, :], ct[2:3, :]], axis=0)
        words = _pack_pairs(los, his)
        ptab_ref[pl.ds(8 * h, 8), :] = jnp.concatenate(
            [words, jnp.zeros((2, 128), jnp.int32)], axis=0)

    n_mp = lm_ref.shape[0]
    for h in range(n_mp // 128):
        lt = _transpose(lm_ref[pl.ds(128 * h, 128), :])
        los = jnp.concatenate([lt[0:1, :], lt[2:3, :]], axis=0)
        his = jnp.concatenate([lt[1:2, :], jnp.zeros((1, 128),
                                                     jnp.float32)], axis=0)
        ltab_ref[pl.ds(2 * h, 2), :] = _pack_pairs(los, his)

    km = k_ref[:]
    aux_ref[:] = jnp.concatenate(
        [
            jnp.broadcast_to(km[0:1, 0:1], (1, 128)),
            jnp.broadcast_to(km[1:2, 1:2], (1, 128)),
            jnp.broadcast_to(km[0:1, 2:3], (1, 128)),
            jnp.broadcast_to(km[1:2, 2:3], (1, 128)),
            jnp.zeros((4, 128), jnp.float32),
        ],
        axis=0,
    )


def _make_tables(quats_log, camera_position, k_mat, landmarks):
    n_kf = quats_log.shape[0]
    n_mp = landmarks.shape[0]
    return pl.pallas_call(
        _tables_tc_body,
        out_shape=(
            jax.ShapeDtypeStruct((n_kf // 16, 128), jnp.int32),
            jax.ShapeDtypeStruct((8, 128), jnp.float32),
            jax.ShapeDtypeStruct((n_mp // 64, 128), jnp.int32),
        ),
    )(quats_log, camera_position, k_mat, landmarks)


def _make_sc_project(m_obs, n_kf, n_mp):
    chunk = m_obs // _NW
    steps = chunk // _L
    mesh = plsc.VectorSubcoreMesh(core_axis_name="c", subcore_axis_name="s")

    @functools.partial(
        pl.kernel,
        out_type=(jax.ShapeDtypeStruct((m_obs,), jnp.float32),
                  jax.ShapeDtypeStruct((m_obs,), jnp.float32)),
        mesh=mesh,
        compiler_params=pltpu.CompilerParams(needs_layout_passes=False),
        scratch_types=[
            pltpu.VMEM((n_kf // 16, 128), jnp.int32),
            pltpu.VMEM((8, 128), jnp.float32),
            pltpu.VMEM((n_mp // 64, 128), jnp.int32),
            pltpu.VMEM((chunk,), jnp.int32),
            pltpu.VMEM((chunk,), jnp.int32),
            pltpu.VMEM((chunk,), jnp.float32),
            pltpu.VMEM((chunk,), jnp.float32),
            pltpu.SemaphoreType.DMA,
        ],
    )
    def sc_project(ptab_hbm, aux_hbm, ltab_hbm, fid_hbm, pid_hbm, u_hbm, v_hbm,
                   ptab_v, aux_v, ltab_v, fid_v, pid_v, u_v, v_v, sem):
        wid = lax.axis_index("s") * _NC + lax.axis_index("c")
        base = wid * chunk
        cp1 = pltpu.async_copy(ptab_hbm, ptab_v, sem)
        cp2 = pltpu.async_copy(aux_hbm, aux_v, sem)
        cp3 = pltpu.async_copy(ltab_hbm, ltab_v, sem)
        cp4 = pltpu.async_copy(fid_hbm.at[pl.ds(base, chunk)], fid_v, sem)
        cp5 = pltpu.async_copy(pid_hbm.at[pl.ds(base, chunk)], pid_v, sem)
        cp1.wait()
        cp2.wait()
        cp3.wait()
        cp4.wait()
        cp5.wait()

        fxv = aux_v[0, pl.ds(0, _L)]
        fyv = aux_v[1, pl.ds(0, _L)]
        cxv = aux_v[2, pl.ds(0, _L)]
        cyv = aux_v[3, pl.ds(0, _L)]
        himask = jnp.full((_L,), jnp.int32(-65536))

        def cvec(k):
            return jnp.full((_L,), k, jnp.int32)

        def lo(w):
            return plsc.bitcast(w << 16, jnp.float32)

        def hi(w):
            return plsc.bitcast(w & himask, jnp.float32)

        @plsc.parallel_loop(0, steps, unroll=4)
        def step(i):
            off = i * _L
            fid = fid_v[pl.ds(off, _L)]
            pid = pid_v[pl.ds(off, _L)]
            fhi = (fid >> 7) << 3
            flo = fid & 127
            phi = (pid >> 7) << 1
            plo = pid & 127
            w0 = plsc.load_gather(ptab_v, [fhi + cvec(0), flo])
            w1 = plsc.load_gather(ptab_v, [fhi + cvec(1), flo])
            w2 = plsc.load_gather(ptab_v, [fhi + cvec(2), flo])
            w3 = plsc.load_gather(ptab_v, [fhi + cvec(3), flo])
            w4 = plsc.load_gather(ptab_v, [fhi + cvec(4), flo])
            w5 = plsc.load_gather(ptab_v, [fhi + cvec(5), flo])
            v0 = plsc.load_gather(ltab_v, [phi + cvec(0), plo])
            v1 = plsc.load_gather(ltab_v, [phi + cvec(1), plo])
            r00, r01 = lo(w0), hi(w0)
            r02, r10 = lo(w1), hi(w1)
            r11, r12 = lo(w2), hi(w2)
            r20, r21 = lo(w3), hi(w3)
            r22, tx = lo(w4), hi(w4)
            ty, tz = lo(w5), hi(w5)
            px, py = lo(v0), hi(v0)
            pz = lo(v1)
            xc = r00 * px + r01 * py + r02 * pz + tx
            yc = r10 * px + r11 * py + r12 * pz + ty
            zc = r20 * px + r21 * py + r22 * pz + tz
            s = jnp.where(jnp.abs(zc) > 1e-8, 1.0 / zc, jnp.ones_like(zc))
            u_v[pl.ds(off, _L)] = (xc * s) * fxv + cxv
            v_v[pl.ds(off, _L)] = (yc * s) * fyv + cyv

        pltpu.sync_copy(u_v, u_hbm.at[pl.ds(base, chunk)])
        pltpu.sync_copy(v_v, v_hbm.at[pl.ds(base, chunk)])

    return sc_project


def kernel(QuatsLog, CameraPosition, Landmarks, K, frame_id, point_id):
    ptab, aux, ltab = _make_tables(QuatsLog, CameraPosition, K, Landmarks)
    m_obs = frame_id.shape[0]
    fid = frame_id.reshape(m_obs)
    pid = point_id.reshape(m_obs)
    sc_project = _make_sc_project(m_obs, QuatsLog.shape[0], Landmarks.shape[0])
    u, v = sc_project(ptab, aux, ltab, fid, pid)
    return jnp.concatenate([u.reshape(m_obs, 1), v.reshape(m_obs, 1)], axis=1)
